# baseline (speedup 1.0000x reference)
import functools

import jax
import jax.numpy as jnp
from jax import lax
from jax.experimental import pallas as pl
from jax.experimental.pallas import tpu as pltpu
from jax.experimental.pallas import tpu_sc as plsc

_NC = 2
_NS = 16
_NW = _NC * _NS
_L = 16

_BAND = 128


@functools.lru_cache(maxsize=None)
def _make_gather(T, BATCH, V2, D):
    mesh = plsc.VectorSubcoreMesh(core_axis_name="c", subcore_axis_name="s")

    @functools.partial(
        pl.kernel,
        out_type=jax.ShapeDtypeStruct((T, D, BATCH), jnp.float32),
        mesh=mesh,
        scratch_types=[
            pltpu.VMEM((8, _BAND), jnp.int32),
            pltpu.VMEM((2, _BAND), jnp.int32),
            pltpu.VMEM((2, _BAND), jnp.int32),
            pltpu.VMEM((2, _BAND, 2 * D), jnp.float32),
            pltpu.VMEM((D, _BAND), jnp.float32),
            pltpu.SemaphoreType.DMA,
        ],
        compiler_params=pltpu.CompilerParams(
            use_tc_tiling_on_sc=True, needs_layout_passes=False),
    )
    def k(pairs_hbm, idx_hbm, out_hbm, idx_v, pidx_v, hcol_v, rows_v,
          trans_v, gsem):
        wid = lax.axis_index("s") * _NC + lax.axis_index("c")
        bcol = pl.multiple_of(wid * _BAND, _BAND)

        iota = lax.iota(jnp.int32, _L)

        def load_octet(t):
            row = pl.multiple_of((t // 8) * 8, 8)
            pltpu.sync_copy(idx_hbm.at[pl.ds(row, 8), pl.ds(bcol, _BAND)],
                            idx_v)

        def stage_unit(u, buf):
            ts = lax.rem(u, 8)
            for g in range(_BAND // _L):
                v = idx_v[ts, pl.ds(g * _L, _L)]
                pidx_v[buf, pl.ds(g * _L, _L)] = v >> 1
                hcol_v[buf, pl.ds(g * _L, _L)] = (v & 1) * D

        def gather_start(buf):
            pltpu.async_copy(pairs_hbm.at[pidx_v.at[buf]], rows_v.at[buf],
                             gsem)

        def gather_wait(buf):
            pltpu.make_async_copy(pairs_hbm.at[pidx_v.at[buf]],
                                  rows_v.at[buf], gsem).wait()

        def transpose_store(t, buf):
            for g in range(_BAND // _L):
                jb = g * _L
                col0 = hcol_v[buf, pl.ds(jb, _L)]
                rowv = iota + jb

                def dstep(d, c):
                    for du in range(4):
                        val = plsc.load_gather(
                            rows_v.at[buf], [rowv, col0 + (4 * d + du)])
                        trans_v[4 * d + du, pl.ds(jb, _L)] = val
                    return c

                lax.fori_loop(0, D // 4, dstep, 0)
            pltpu.sync_copy(trans_v, out_hbm.at[t, :, pl.ds(bcol, _BAND)])

        load_octet(0)
        stage_unit(0, 0)
        gather_start(0)

        def body(u, carry):
            buf = lax.rem(u, 2)
            nbuf = 1 - buf

            def launch_next():
                pl.when(lax.rem(u, 8) == 7)(lambda: load_octet(u + 1))
                stage_unit(u + 1, nbuf)
                gather_start(nbuf)

            pl.when(u + 1 < T)(launch_next)

            gather_wait(buf)
            transpose_store(u, buf)
            return carry

        lax.fori_loop(0, T, body, 0)

    return k


def kernel(words, weight):
    BATCH, T = words.shape
    V, D = weight.shape
    pairs = weight.reshape(V // 2, 2 * D)
    idx_t = words.T.astype(jnp.int32)
    out3 = _make_gather(T, BATCH, V // 2, D)(pairs, idx_t)
    return jnp.transpose(out3, (2, 0, 1))
